# baseline (speedup 1.0000x reference)
import functools

import jax
import jax.numpy as jnp
from jax.experimental import pallas as pl
from jax.experimental.pallas import tpu as pltpu

_DN = (((1,), (1,)), ((), ()))


def _qgps_body(chunks, s_hbm, e_ref, o_ref, s_vmem, sems):
    _CHUNKS = chunks
    rows = s_vmem.shape[0] // _CHUNKS
    for i in range(_CHUNKS):
        pltpu.make_async_copy(
            s_hbm.at[pl.ds(i * rows, rows), :],
            s_vmem.at[pl.ds(i * rows, rows), :],
            sems.at[i],
        ).start()

    e0 = e_ref[0]
    e1 = e_ref[1]
    t0 = jnp.maximum(jnp.log(jnp.abs(e0)), -1e4)
    t1 = jnp.maximum(jnp.log(jnp.abs(e1)), -1e4)
    n0 = (e0 < 0).astype(jnp.float32)
    n1 = (e1 < 0).astype(jnp.float32)
    rhs = jnp.concatenate([t1 - t0, n1 - n0], axis=0)
    ref0 = jnp.concatenate([t0, n0], axis=0)
    ones = jnp.ones((1, ref0.shape[1]), jnp.float32)
    base = jax.lax.dot_general(ones, ref0, _DN,
                               preferred_element_type=jnp.float32)
    n = e_ref.shape[1]
    for i in range(_CHUNKS):
        pltpu.make_async_copy(
            s_hbm.at[pl.ds(i * rows, rows), :],
            s_vmem.at[pl.ds(i * rows, rows), :],
            sems.at[i],
        ).wait()
        sf = s_vmem[pl.ds(i * rows, rows), :].astype(jnp.float32)
        acc = base + jax.lax.dot_general(sf, rhs, _DN,
                                         preferred_element_type=jnp.float32)
        logp = acc[:, :n]
        negs = acc[:, n:]
        sign = 1.0 - 2.0 * (negs - 2.0 * jnp.floor(negs * 0.5))
        psi = sign * jnp.exp(logp)
        o_ref[pl.ds(i * rows, rows), :] = jnp.sum(psi, axis=1, keepdims=True)


def kernel(inputs, eps):
    if inputs.ndim == 1:
        inputs = jnp.expand_dims(inputs, axis=0)
    B, L = inputs.shape
    N = eps.shape[1]
    chunks = 4 if B % 32 == 0 else 1
    out = pl.pallas_call(
        functools.partial(_qgps_body, chunks),
        in_specs=[
            pl.BlockSpec(memory_space=pl.ANY),
            pl.BlockSpec((2, N, L), lambda: (0, 0, 0)),
        ],
        out_specs=pl.BlockSpec((B, 1), lambda: (0, 0)),
        out_shape=jax.ShapeDtypeStruct((B, 1), jnp.float32),
        scratch_shapes=[
            pltpu.VMEM((B, L), jnp.int32),
            pltpu.SemaphoreType.DMA((chunks,)),
        ],
    )(inputs, eps)
    return out.reshape(B)
